# Initial kernel scaffold; baseline (speedup 1.0000x reference)
import jax
import jax.numpy as jnp
from jax.experimental import pallas as pl


def kernel(length_query, length_key, position_embeddings):
    raise NotImplementedError("write your pallas kernel here")


# speedup vs baseline: 5.7907x; 5.7907x over previous
import jax
import jax.numpy as jnp
from jax import lax
from jax.experimental import pallas as pl
from jax.experimental.pallas import tpu as pltpu
from jax.experimental.pallas import tpu_sc as plsc

D_A = 64
K_CLIP = 64
L_Q = 2048
L_K = 2048
E_ROWS = 4096

_INFO = plsc.get_sparse_core_info()
NC = _INFO.num_cores
NS = _INFO.num_subcores
LANES = _INFO.num_lanes
NW = NC * NS
ROWS_PER_W = L_Q // NW
E_PER_S = E_ROWS // NS
GCHUNK = 128


def _sc_body(table_hbm, delta_hbm, out_hbm, e_spmem, idx_v, rows_v, delta_v, sem):
    cid = lax.axis_index("c")
    sid = lax.axis_index("s")
    pltpu.sync_copy(delta_hbm, delta_v)
    dvec = delta_v[...]
    lanes = lax.iota(jnp.int32, LANES)

    for rnd in range(E_PER_S // GCHUNK):
        base = sid * E_PER_S + rnd * GCHUNK
        for c in range(GCHUNK // LANES):
            t = base + c * LANES + lanes
            idx = jnp.clip(t - (L_Q - 1) + dvec, -K_CLIP, K_CLIP) + K_CLIP
            idx_v[pl.ds(c * LANES, LANES)] = idx
        pltpu.async_copy(table_hbm.at[idx_v], rows_v, sem).wait()
        pltpu.sync_copy(rows_v, e_spmem.at[pl.ds(base, GCHUNK)])
    plsc.subcore_barrier()

    row0 = (cid * NS + sid) * ROWS_PER_W
    for r in range(ROWS_PER_W):
        i = row0 + r
        start = (L_Q - 1) - i
        pltpu.sync_copy(e_spmem.at[pl.ds(start, L_K)], out_hbm.at[i])


def kernel(length_query, length_key, position_embeddings):
    delta = jnp.full(
        (LANES,),
        jnp.asarray(length_key, jnp.int32) - jnp.asarray(length_query, jnp.int32),
        jnp.int32,
    )
    run = pl.kernel(
        _sc_body,
        out_type=jax.ShapeDtypeStruct((L_Q, L_K, D_A), jnp.float32),
        mesh=plsc.VectorSubcoreMesh(core_axis_name="c", subcore_axis_name="s"),
        scratch_types=[
            pltpu.VMEM_SHARED((E_ROWS, D_A), jnp.float32),
            pltpu.VMEM((GCHUNK,), jnp.int32),
            pltpu.VMEM((GCHUNK, D_A), jnp.float32),
            pltpu.VMEM((LANES,), jnp.int32),
            pltpu.SemaphoreType.DMA,
        ],
        compiler_params=pltpu.CompilerParams(use_tc_tiling_on_sc=False),
    )
    return run(position_embeddings.astype(jnp.float32), delta)
